# baseline (speedup 1.0000x reference)
import jax
import jax.numpy as jnp
from jax.experimental import pallas as pl
from jax.experimental.pallas import tpu as pltpu

_NEG_BIG = -1e9
_BB = 2


def _pool_kernel(x_ref, m_ref, w_ref, b_ref, u_ref, o_ref, xb_ref):
    def scores_batch(bi):
        xb = x_ref[bi].astype(jnp.bfloat16)
        xb_ref[bi] = xb
        uit = jnp.tanh(
            jax.lax.dot_general(
                xb, w_ref[...], (((1,), (0,)), ((), ())),
                preferred_element_type=jnp.float32,
            )
            + b_ref[...]
        ).astype(jnp.bfloat16)
        return jax.lax.dot_general(
            u_ref[...], uit, (((1,), (1,)), ((), ())),
            preferred_element_type=jnp.float32,
        )

    def finish_batch(bi, scores):
        mrow = m_ref[bi].astype(jnp.float32)
        scores = scores + (1.0 - mrow) * _NEG_BIG
        smax = jnp.max(scores, axis=1, keepdims=True)
        p = jnp.exp(scores - smax)
        s = jnp.sum(p, axis=1, keepdims=True)
        ctx = jax.lax.dot_general(
            p.astype(jnp.bfloat16), xb_ref[bi], (((1,), (0,)), ((), ())),
            preferred_element_type=jnp.float32,
        )
        o_ref[bi] = ctx * (1.0 / s)

    all_scores = [scores_batch(bi) for bi in range(_BB)]
    for bi in range(_BB):
        finish_batch(bi, all_scores[bi])


def kernel(inputs, mask, w, b, u):
    B, T, D = inputs.shape
    U = w.shape[1]
    mask3 = mask.reshape(B, 1, T)
    b_row = b.reshape(1, U)
    u_row = u.reshape(1, U).astype(jnp.bfloat16)
    w16 = w.astype(jnp.bfloat16)

    out = pl.pallas_call(
        _pool_kernel,
        grid=(B // _BB,),
        in_specs=[
            pl.BlockSpec((_BB, T, D), lambda i: (i, 0, 0)),
            pl.BlockSpec((_BB, 1, T), lambda i: (i, 0, 0)),
            pl.BlockSpec((D, U), lambda i: (0, 0)),
            pl.BlockSpec((1, U), lambda i: (0, 0)),
            pl.BlockSpec((1, U), lambda i: (0, 0)),
        ],
        out_specs=pl.BlockSpec((_BB, 1, D), lambda i: (i, 0, 0)),
        out_shape=jax.ShapeDtypeStruct((B, 1, D), jnp.float32),
        scratch_shapes=[pltpu.VMEM((_BB, T, D), jnp.bfloat16)],
        compiler_params=pltpu.CompilerParams(
            dimension_semantics=("arbitrary",),
            vmem_limit_bytes=48 * 1024 * 1024,
        ),
        name="attention_pooling",
    )(inputs, mask3, w16, b_row, u_row)
    return out.reshape(B, D)


# speedup vs baseline: 1.1031x; 1.1031x over previous
import jax
import jax.numpy as jnp
from jax.experimental import pallas as pl
from jax.experimental.pallas import tpu as pltpu

_NEG_BIG = -1e9
_BB = 4


def _pool_kernel(x_ref, m_ref, w_ref, b_ref, u_ref, o_ref, xb_ref):
    def scores_batch(bi):
        xb = x_ref[bi].astype(jnp.bfloat16)
        xb_ref[bi] = xb
        uit = jnp.tanh(
            jax.lax.dot_general(
                xb, w_ref[...], (((1,), (0,)), ((), ())),
                preferred_element_type=jnp.float32,
            )
            + b_ref[...]
        ).astype(jnp.bfloat16)
        return jax.lax.dot_general(
            u_ref[...], uit, (((1,), (1,)), ((), ())),
            preferred_element_type=jnp.float32,
        )

    def finish_batch(bi, scores):
        mrow = m_ref[bi].astype(jnp.float32)
        scores = scores + (1.0 - mrow) * _NEG_BIG
        smax = jnp.max(scores, axis=1, keepdims=True)
        p = jnp.exp(scores - smax)
        s = jnp.sum(p, axis=1, keepdims=True)
        ctx = jax.lax.dot_general(
            p.astype(jnp.bfloat16), xb_ref[bi], (((1,), (0,)), ((), ())),
            preferred_element_type=jnp.float32,
        )
        o_ref[bi] = ctx * (1.0 / s)

    all_scores = [scores_batch(bi) for bi in range(_BB)]
    for bi in range(_BB):
        finish_batch(bi, all_scores[bi])


def kernel(inputs, mask, w, b, u):
    B, T, D = inputs.shape
    U = w.shape[1]
    mask3 = mask.reshape(B, 1, T)
    b_row = b.reshape(1, U)
    u_row = u.reshape(1, U).astype(jnp.bfloat16)
    w16 = w.astype(jnp.bfloat16)

    out = pl.pallas_call(
        _pool_kernel,
        grid=(B // _BB,),
        in_specs=[
            pl.BlockSpec((_BB, T, D), lambda i: (i, 0, 0)),
            pl.BlockSpec((_BB, 1, T), lambda i: (i, 0, 0)),
            pl.BlockSpec((D, U), lambda i: (0, 0)),
            pl.BlockSpec((1, U), lambda i: (0, 0)),
            pl.BlockSpec((1, U), lambda i: (0, 0)),
        ],
        out_specs=pl.BlockSpec((_BB, 1, D), lambda i: (i, 0, 0)),
        out_shape=jax.ShapeDtypeStruct((B, 1, D), jnp.float32),
        scratch_shapes=[pltpu.VMEM((_BB, T, D), jnp.bfloat16)],
        compiler_params=pltpu.CompilerParams(
            dimension_semantics=("arbitrary",),
            vmem_limit_bytes=48 * 1024 * 1024,
        ),
        name="attention_pooling",
    )(inputs, mask3, w16, b_row, u_row)
    return out.reshape(B, D)
